# Initial kernel scaffold; baseline (speedup 1.0000x reference)
import jax
import jax.numpy as jnp
from jax.experimental import pallas as pl


def kernel(pred_logits, pred_boxes, target_sizes, positive_map, num_select):
    raise NotImplementedError("write your pallas kernel here")


# speedup vs baseline: 6.5087x; 6.5087x over previous
import functools
import jax
import jax.numpy as jnp
from jax import lax
from jax.experimental import pallas as pl
from jax.experimental.pallas import tpu as pltpu
from jax.experimental.pallas import tpu_sc as plsc

B, Q, T, C = 32, 900, 256, 91
QP, CP = 904, 128
N = QP * CP
K = 300
L = 16
NC, NS = 2, 16
NCHUNK = 8
CH = N // NCHUNK
CHV = CH // L
HB = 2048
FB = 256
CS = 20
FS = 12
CAP = 2048
MP = 320
KOUT = 304


def _prob_kernel(sig_ref, pmt_ref, out_ref):
    p = jnp.dot(sig_ref[0], pmt_ref[...], preferred_element_type=jnp.float32)
    qi = lax.broadcasted_iota(jnp.int32, (QP, CP), 0)
    out_ref[0] = jnp.where(qi < Q, p, 0.0)


def _tc_prob(sig_pad, pmt):
    return pl.pallas_call(
        _prob_kernel,
        grid=(B,),
        in_specs=[
            pl.BlockSpec((1, QP, T), lambda b: (b, 0, 0)),
            pl.BlockSpec((T, CP), lambda b: (0, 0)),
        ],
        out_specs=pl.BlockSpec((1, QP, CP), lambda b: (b, 0, 0)),
        out_shape=jax.ShapeDtypeStruct((B, QP, CP), jnp.float32),
    )(sig_pad, pmt)


def _sc_body(prob_hbm, bxf_hbm, scale_hbm,
             scores_hbm, labels_hbm, boxes_hbm,
             dbuf0, dbuf1, hist, histf, candk, candi, cand2k, cand2i,
             score_st, label_st, box_st, bx, scv, msc,
             sem0, sem1, semb):
    wid = lax.axis_index("c") * NS + lax.axis_index("s")
    iota = lax.iota(jnp.int32, L)
    ones = jnp.ones((L,), jnp.int32)
    zeros_i = jnp.zeros((L,), jnp.int32)

    box_cp = pltpu.make_async_copy(bxf_hbm.at[wid], bx, semb)
    box_cp.start()
    pltpu.sync_copy(scale_hbm.at[wid], scv)

    def _z(i, _):
        hist[pl.ds(i * L, L)] = zeros_i
        return 0
    lax.fori_loop(0, HB, _z, 0)

    def _start(k, buf, sem):
        pltpu.make_async_copy(prob_hbm.at[wid, pl.ds(k * CH, CH)], buf, sem).start()

    def _wait(k, buf, sem):
        pltpu.make_async_copy(prob_hbm.at[wid, pl.ds(k * CH, CH)], buf, sem).wait()

    def _hist_chunk(buf):
        def inner(v, _):
            x = buf[pl.ds(v * L, L)]
            key = plsc.bitcast(x, jnp.int32)
            bins = lax.shift_right_logical(key, CS)
            plsc.addupdate_scatter(hist, [bins * L + iota], ones)
            return 0
        lax.fori_loop(0, CHV, inner, 0)

    _start(0, dbuf0, sem0)

    def _p1_outer(i, _):
        for par in range(2):
            k = i * 2 + par
            buf, sem = (dbuf0, sem0) if par == 0 else (dbuf1, sem1)
            nbuf, nsem = (dbuf1, sem1) if par == 0 else (dbuf0, sem0)

            @pl.when(k + 1 < NCHUNK)
            def _():
                _start(k + 1, nbuf, nsem)

            _wait(k, buf, sem)
            _hist_chunk(buf)
        return 0
    lax.fori_loop(0, NCHUNK // 2, _p1_outer, 0)

    def _row_sum(h, b_):
        return jnp.sum(h[pl.ds(b_ * L, L)])

    def _c_cond(c):
        b_, n = c
        return (n + _row_sum(hist, b_) < K) & (b_ > 0)

    def _c_body(c):
        b_, n = c
        return b_ - 1, n + _row_sum(hist, b_)

    Tb, n_hi = lax.while_loop(_c_cond, _c_body, (HB - 1, jnp.int32(0)))
    thrv = jnp.full((L,), lax.bitcast_convert_type(Tb << CS, jnp.float32))

    def _collect_chunk(k, buf, off_v):
        def inner(v, off):
            x = buf[pl.ds(v * L, L)]
            keep = x >= thrv
            cs = plsc.cumsum(jnp.where(keep, 1, 0))
            pos = jnp.minimum(off + cs - 1, CAP - 1)
            gv = (k * CH + v * L) + iota
            plsc.store_scatter(candk, [pos], x, mask=keep)
            plsc.store_scatter(candi, [pos], gv, mask=keep)
            return off + plsc.all_reduce_population_count(keep)
        return lax.fori_loop(0, CHV, inner, off_v)

    _start(0, dbuf0, sem0)

    def _p2_outer(i, off_v):
        for par in range(2):
            k = i * 2 + par
            buf, sem = (dbuf0, sem0) if par == 0 else (dbuf1, sem1)
            nbuf, nsem = (dbuf1, sem1) if par == 0 else (dbuf0, sem0)

            @pl.when(k + 1 < NCHUNK)
            def _():
                _start(k + 1, nbuf, nsem)

            _wait(k, buf, sem)
            off_v = _collect_chunk(k, buf, off_v)
        return off_v
    off_v = lax.fori_loop(0, NCHUNK // 2, _p2_outer, zeros_i)

    msc[pl.ds(0, L)] = off_v
    m1 = jnp.minimum(msc[pl.ds(0, L)][0], CAP)
    trips = (m1 + (L - 1)) // L

    def _zf(i, _):
        histf[pl.ds(i * L, L)] = zeros_i
        return 0
    lax.fori_loop(0, FB, _zf, 0)

    upv = jnp.full((L,), lax.bitcast_convert_type((Tb + 1) << CS, jnp.float32))

    def _fh(v, _):
        x = candk[pl.ds(v * L, L)]
        lanev = (v * L + iota) < m1
        mk = (x < upv) & lanev
        fb = lax.shift_right_logical(plsc.bitcast(x, jnp.int32), FS) & (FB - 1)
        plsc.addupdate_scatter(histf, [fb * L + iota], ones, mask=mk)
        return 0
    lax.fori_loop(0, trips, _fh, 0)

    k2 = K - n_hi

    def _f_cond(c):
        b_, n = c
        return (n + _row_sum(histf, b_) < k2) & (b_ > 0)

    def _f_body(c):
        b_, n = c
        return b_ - 1, n + _row_sum(histf, b_)

    T2, _n2 = lax.while_loop(_f_cond, _f_body, (FB - 1, jnp.int32(0)))
    thr2v = jnp.full((L,), lax.bitcast_convert_type((Tb << CS) | (T2 << FS),
                                                    jnp.float32))

    ninf = jnp.full((L,), -jnp.inf, jnp.float32)
    for a in range(MP // L):
        cand2k[pl.ds(a * L, L)] = ninf
        cand2i[pl.ds(a * L, L)] = zeros_i

    def _cp(v, off):
        x = candk[pl.ds(v * L, L)]
        gi = candi[pl.ds(v * L, L)]
        lanev = (v * L + iota) < m1
        keep = (x >= thr2v) & lanev
        cs = plsc.cumsum(jnp.where(keep, 1, 0))
        pos = jnp.minimum(off + cs - 1, MP - 1)
        plsc.store_scatter(cand2k, [pos], x, mask=keep)
        plsc.store_scatter(cand2i, [pos], gi, mask=keep)
        return off + plsc.all_reduce_population_count(keep)
    lax.fori_loop(0, trips, _cp, zeros_i)

    AC = MP // L
    ranks = []
    for half in range(2):
        ka = [cand2k[pl.ds(a * L, L)] for a in range(half * AC // 2,
                                                     (half + 1) * AC // 2)]

        def _rank(j, cnts, ka=ka, half=half):
            kj = jnp.full((L,), cand2k[pl.ds(j, L)][0])
            jv = jnp.full((L,), j)
            out = []
            for ai, a in enumerate(range(half * AC // 2, (half + 1) * AC // 2)):
                gt = kj > ka[ai]
                eq = (kj == ka[ai]) & ((jv - a * L) < iota)
                out.append(cnts[ai] + jnp.where(gt | eq, 1, 0))
            return tuple(out)

        cnts = lax.fori_loop(0, MP, _rank, tuple(zeros_i for _ in range(AC // 2)))
        ranks.extend(cnts)

    box_cp.wait()
    sv = scv[pl.ds(0, L)]
    wimg = jnp.full((L,), sv[0])
    himg = jnp.full((L,), sv[1])
    for a in range(AC):
        r = ranks[a]
        ka = cand2k[pl.ds(a * L, L)]
        gi = cand2i[pl.ds(a * L, L)]
        cls = gi & (CP - 1)
        q4 = lax.shift_right_logical(gi, 7) * 4
        plsc.store_scatter(score_st, [r], ka)
        plsc.store_scatter(label_st, [r], cls)
        cx = plsc.load_gather(bx, [q4])
        cy = plsc.load_gather(bx, [q4 + 1])
        w = plsc.load_gather(bx, [q4 + 2])
        h = plsc.load_gather(bx, [q4 + 3])
        r4 = r * 4
        plsc.store_scatter(box_st, [r4], (cx - 0.5 * w) * wimg)
        plsc.store_scatter(box_st, [r4 + 1], (cy - 0.5 * h) * himg)
        plsc.store_scatter(box_st, [r4 + 2], (cx + 0.5 * w) * wimg)
        plsc.store_scatter(box_st, [r4 + 3], (cy + 0.5 * h) * himg)

    pltpu.sync_copy(score_st.at[pl.ds(0, KOUT)], scores_hbm.at[wid])
    pltpu.sync_copy(label_st.at[pl.ds(0, KOUT)], labels_hbm.at[wid])
    pltpu.sync_copy(box_st.at[pl.ds(0, KOUT * 4)], boxes_hbm.at[wid])


@functools.lru_cache(maxsize=None)
def _sc_topk_call():
    return pl.kernel(
        _sc_body,
        out_type=[
        jax.ShapeDtypeStruct((B, KOUT), jnp.float32),
        jax.ShapeDtypeStruct((B, KOUT), jnp.int32),
        jax.ShapeDtypeStruct((B, KOUT * 4), jnp.float32),
    ],
    mesh=plsc.VectorSubcoreMesh(core_axis_name="c", subcore_axis_name="s",
                                num_cores=NC, num_subcores=NS),
    compiler_params=pltpu.CompilerParams(needs_layout_passes=False,
                                         use_tc_tiling_on_sc=False),
    scratch_types=[
        pltpu.VMEM((CH,), jnp.float32),
        pltpu.VMEM((CH,), jnp.float32),
        pltpu.VMEM((HB * L,), jnp.int32),
        pltpu.VMEM((FB * L,), jnp.int32),
        pltpu.VMEM((CAP,), jnp.float32),
        pltpu.VMEM((CAP,), jnp.int32),
        pltpu.VMEM((MP + L,), jnp.float32),
        pltpu.VMEM((MP,), jnp.int32),
        pltpu.VMEM((MP,), jnp.float32),
        pltpu.VMEM((MP,), jnp.int32),
        pltpu.VMEM((MP * 4,), jnp.float32),
        pltpu.VMEM((Q * 4,), jnp.float32),
        pltpu.VMEM((L,), jnp.float32),
        pltpu.VMEM((L,), jnp.int32),
        pltpu.SemaphoreType.DMA,
        pltpu.SemaphoreType.DMA,
        pltpu.SemaphoreType.DMA,
    ],
)


def kernel(pred_logits, pred_boxes, target_sizes, positive_map, num_select):
    del num_select
    sig = jax.nn.sigmoid(pred_logits)
    sig_pad = jnp.pad(sig, ((0, 0), (0, QP - Q), (0, 0)))
    pmt = jnp.pad(positive_map, ((0, CP - C), (0, 0))).T
    prob_flat = _tc_prob(sig_pad, pmt).reshape(B, N)

    bxf = pred_boxes.reshape(B, Q * 4)
    ts = target_sizes.astype(jnp.float32)
    scale16 = jnp.tile(jnp.stack([ts[:, 1], ts[:, 0]], axis=1), (1, 8))

    scores_p, labels_p, boxes_p = _sc_topk_call()(prob_flat, bxf, scale16)
    return (scores_p[:, :K], labels_p[:, :K],
            boxes_p[:, : K * 4].reshape(B, K, 4))


# speedup vs baseline: 9.9768x; 1.5328x over previous
import functools
import jax
import jax.numpy as jnp
from jax import lax
from jax.experimental import pallas as pl
from jax.experimental.pallas import tpu as pltpu
from jax.experimental.pallas import tpu_sc as plsc

B, Q, T, C = 32, 900, 256, 91
QP, CP = 904, 128
N = QP * CP
K = 300
L = 16
NC, NS = 2, 16
NCHUNK = 8
CH = N // NCHUNK
CHV = CH // L
HB = 2048
FB = 256
CS = 20
FS = 12
CAP = 2048
MP = 320
KOUT = 304


def _prob_kernel(sig_ref, pmt_ref, out_ref):
    p = jnp.dot(sig_ref[0], pmt_ref[...], preferred_element_type=jnp.float32)
    qi = lax.broadcasted_iota(jnp.int32, (QP, CP), 0)
    out_ref[0] = jnp.where(qi < Q, p, 0.0)


def _tc_prob(sig_pad, pmt):
    return pl.pallas_call(
        _prob_kernel,
        grid=(B,),
        in_specs=[
            pl.BlockSpec((1, QP, T), lambda b: (b, 0, 0)),
            pl.BlockSpec((T, CP), lambda b: (0, 0)),
        ],
        out_specs=pl.BlockSpec((1, QP, CP), lambda b: (b, 0, 0)),
        out_shape=jax.ShapeDtypeStruct((B, QP, CP), jnp.float32),
    )(sig_pad, pmt)


def _sc_body(prob_hbm, bxf_hbm, scale_hbm,
             scores_hbm, labels_hbm, boxes_hbm,
             dbuf0, dbuf1, hist, histf, candk, candi, cand2k, cand2i,
             score_st, label_st, box_st, bx, scv, msc,
             sem0, sem1, semb):
    wid = lax.axis_index("c") * NS + lax.axis_index("s")
    iota = lax.iota(jnp.int32, L)
    ones = jnp.ones((L,), jnp.int32)
    zeros_i = jnp.zeros((L,), jnp.int32)

    box_cp = pltpu.make_async_copy(bxf_hbm.at[wid], bx, semb)
    box_cp.start()
    pltpu.sync_copy(scale_hbm.at[wid], scv)

    @plsc.parallel_loop(0, HB, unroll=8)
    def _(i):
        hist[pl.ds(i * L, L)] = zeros_i

    def _start(k, buf, sem):
        pltpu.make_async_copy(prob_hbm.at[wid, pl.ds(k * CH, CH)], buf, sem).start()

    def _wait(k, buf, sem):
        pltpu.make_async_copy(prob_hbm.at[wid, pl.ds(k * CH, CH)], buf, sem).wait()

    def _hist_chunk(buf):
        @plsc.parallel_loop(0, CHV, unroll=8)
        def _(v):
            x = buf[pl.ds(v * L, L)]
            key = plsc.bitcast(x, jnp.int32)
            bins = lax.shift_right_logical(key, CS)
            plsc.addupdate_scatter(hist, [bins * L + iota], ones)

    _start(0, dbuf0, sem0)

    def _p1_outer(i, _):
        for par in range(2):
            k = i * 2 + par
            buf, sem = (dbuf0, sem0) if par == 0 else (dbuf1, sem1)
            nbuf, nsem = (dbuf1, sem1) if par == 0 else (dbuf0, sem0)

            @pl.when(k + 1 < NCHUNK)
            def _():
                _start(k + 1, nbuf, nsem)

            _wait(k, buf, sem)
            _hist_chunk(buf)
        return 0
    lax.fori_loop(0, NCHUNK // 2, _p1_outer, 0)

    def _row_sum(h, b_):
        return jnp.sum(h[pl.ds(b_ * L, L)])

    def _c_cond(c):
        b_, n = c
        return (n + _row_sum(hist, b_) < K) & (b_ > 0)

    def _c_body(c):
        b_, n = c
        return b_ - 1, n + _row_sum(hist, b_)

    Tb, n_hi = lax.while_loop(_c_cond, _c_body, (HB - 1, jnp.int32(0)))
    thrv = jnp.full((L,), lax.bitcast_convert_type(Tb << CS, jnp.float32))

    def _collect_chunk(k, buf, off_v):
        def inner(v, off):
            x = buf[pl.ds(v * L, L)]
            keep = x >= thrv
            cs = plsc.cumsum(jnp.where(keep, 1, 0))
            pos = jnp.minimum(off + cs - 1, CAP - 1)
            gv = (k * CH + v * L) + iota
            plsc.store_scatter(candk, [pos], x, mask=keep)
            plsc.store_scatter(candi, [pos], gv, mask=keep)
            return off + plsc.all_reduce_population_count(keep)
        return plsc.parallel_loop(0, CHV, unroll=4, carry=off_v)(inner)

    _start(0, dbuf0, sem0)

    def _p2_outer(i, off_v):
        for par in range(2):
            k = i * 2 + par
            buf, sem = (dbuf0, sem0) if par == 0 else (dbuf1, sem1)
            nbuf, nsem = (dbuf1, sem1) if par == 0 else (dbuf0, sem0)

            @pl.when(k + 1 < NCHUNK)
            def _():
                _start(k + 1, nbuf, nsem)

            _wait(k, buf, sem)
            off_v = _collect_chunk(k, buf, off_v)
        return off_v
    off_v = lax.fori_loop(0, NCHUNK // 2, _p2_outer, zeros_i)

    msc[pl.ds(0, L)] = off_v
    m1 = jnp.minimum(msc[pl.ds(0, L)][0], CAP)
    trips = (m1 + (L - 1)) // L

    @plsc.parallel_loop(0, FB, unroll=8)
    def _(i):
        histf[pl.ds(i * L, L)] = zeros_i

    upv = jnp.full((L,), lax.bitcast_convert_type((Tb + 1) << CS, jnp.float32))

    def _fh(v, _):
        x = candk[pl.ds(v * L, L)]
        lanev = (v * L + iota) < m1
        mk = (x < upv) & lanev
        fb = lax.shift_right_logical(plsc.bitcast(x, jnp.int32), FS) & (FB - 1)
        plsc.addupdate_scatter(histf, [fb * L + iota], ones, mask=mk)
        return 0
    lax.fori_loop(0, trips, _fh, 0)

    k2 = K - n_hi

    def _f_cond(c):
        b_, n = c
        return (n + _row_sum(histf, b_) < k2) & (b_ > 0)

    def _f_body(c):
        b_, n = c
        return b_ - 1, n + _row_sum(histf, b_)

    T2, _n2 = lax.while_loop(_f_cond, _f_body, (FB - 1, jnp.int32(0)))
    thr2v = jnp.full((L,), lax.bitcast_convert_type((Tb << CS) | (T2 << FS),
                                                    jnp.float32))

    ninf = jnp.full((L,), -jnp.inf, jnp.float32)
    for a in range(MP // L):
        cand2k[pl.ds(a * L, L)] = ninf
        cand2i[pl.ds(a * L, L)] = zeros_i

    def _cp(v, off):
        x = candk[pl.ds(v * L, L)]
        gi = candi[pl.ds(v * L, L)]
        lanev = (v * L + iota) < m1
        keep = (x >= thr2v) & lanev
        cs = plsc.cumsum(jnp.where(keep, 1, 0))
        pos = jnp.minimum(off + cs - 1, MP - 1)
        plsc.store_scatter(cand2k, [pos], x, mask=keep)
        plsc.store_scatter(cand2i, [pos], gi, mask=keep)
        return off + plsc.all_reduce_population_count(keep)
    lax.fori_loop(0, trips, _cp, zeros_i)

    AC = MP // L
    ranks = []
    for half in range(2):
        ka = [cand2k[pl.ds(a * L, L)] for a in range(half * AC // 2,
                                                     (half + 1) * AC // 2)]

        def _rank(j, cnts, ka=ka, half=half):
            kj = jnp.full((L,), cand2k[pl.ds(j, L)][0])
            jv = jnp.full((L,), j)
            out = []
            for ai, a in enumerate(range(half * AC // 2, (half + 1) * AC // 2)):
                gt = kj > ka[ai]
                eq = (kj == ka[ai]) & ((jv - a * L) < iota)
                out.append(cnts[ai] + jnp.where(gt | eq, 1, 0))
            return tuple(out)

        cnts = plsc.parallel_loop(0, MP, unroll=2,
                                  carry=tuple(zeros_i for _ in range(AC // 2)))(_rank)
        ranks.extend(cnts)

    box_cp.wait()
    sv = scv[pl.ds(0, L)]
    wimg = jnp.full((L,), sv[0])
    himg = jnp.full((L,), sv[1])
    for a in range(AC):
        r = ranks[a]
        ka = cand2k[pl.ds(a * L, L)]
        gi = cand2i[pl.ds(a * L, L)]
        cls = gi & (CP - 1)
        q4 = lax.shift_right_logical(gi, 7) * 4
        plsc.store_scatter(score_st, [r], ka)
        plsc.store_scatter(label_st, [r], cls)
        cx = plsc.load_gather(bx, [q4])
        cy = plsc.load_gather(bx, [q4 + 1])
        w = plsc.load_gather(bx, [q4 + 2])
        h = plsc.load_gather(bx, [q4 + 3])
        r4 = r * 4
        plsc.store_scatter(box_st, [r4], (cx - 0.5 * w) * wimg)
        plsc.store_scatter(box_st, [r4 + 1], (cy - 0.5 * h) * himg)
        plsc.store_scatter(box_st, [r4 + 2], (cx + 0.5 * w) * wimg)
        plsc.store_scatter(box_st, [r4 + 3], (cy + 0.5 * h) * himg)

    pltpu.sync_copy(score_st.at[pl.ds(0, KOUT)], scores_hbm.at[wid])
    pltpu.sync_copy(label_st.at[pl.ds(0, KOUT)], labels_hbm.at[wid])
    pltpu.sync_copy(box_st.at[pl.ds(0, KOUT * 4)], boxes_hbm.at[wid])


@functools.lru_cache(maxsize=None)
def _sc_topk_call():
    return pl.kernel(
        _sc_body,
        out_type=[
        jax.ShapeDtypeStruct((B, KOUT), jnp.float32),
        jax.ShapeDtypeStruct((B, KOUT), jnp.int32),
        jax.ShapeDtypeStruct((B, KOUT * 4), jnp.float32),
    ],
    mesh=plsc.VectorSubcoreMesh(core_axis_name="c", subcore_axis_name="s",
                                num_cores=NC, num_subcores=NS),
    compiler_params=pltpu.CompilerParams(needs_layout_passes=False,
                                         use_tc_tiling_on_sc=False),
    scratch_types=[
        pltpu.VMEM((CH,), jnp.float32),
        pltpu.VMEM((CH,), jnp.float32),
        pltpu.VMEM((HB * L,), jnp.int32),
        pltpu.VMEM((FB * L,), jnp.int32),
        pltpu.VMEM((CAP,), jnp.float32),
        pltpu.VMEM((CAP,), jnp.int32),
        pltpu.VMEM((MP + L,), jnp.float32),
        pltpu.VMEM((MP,), jnp.int32),
        pltpu.VMEM((MP,), jnp.float32),
        pltpu.VMEM((MP,), jnp.int32),
        pltpu.VMEM((MP * 4,), jnp.float32),
        pltpu.VMEM((Q * 4,), jnp.float32),
        pltpu.VMEM((L,), jnp.float32),
        pltpu.VMEM((L,), jnp.int32),
        pltpu.SemaphoreType.DMA,
        pltpu.SemaphoreType.DMA,
        pltpu.SemaphoreType.DMA,
    ],
)


def kernel(pred_logits, pred_boxes, target_sizes, positive_map, num_select):
    del num_select
    sig = jax.nn.sigmoid(pred_logits)
    sig_pad = jnp.pad(sig, ((0, 0), (0, QP - Q), (0, 0)))
    pmt = jnp.pad(positive_map, ((0, CP - C), (0, 0))).T
    prob_flat = _tc_prob(sig_pad, pmt).reshape(B, N)

    bxf = pred_boxes.reshape(B, Q * 4)
    ts = target_sizes.astype(jnp.float32)
    scale16 = jnp.tile(jnp.stack([ts[:, 1], ts[:, 0]], axis=1), (1, 8))

    scores_p, labels_p, boxes_p = _sc_topk_call()(prob_flat, bxf, scale16)
    return (scores_p[:, :K], labels_p[:, :K],
            boxes_p[:, : K * 4].reshape(B, K, 4))
